# Initial kernel scaffold; baseline (speedup 1.0000x reference)
import jax
import jax.numpy as jnp
from jax.experimental import pallas as pl


def kernel(inputs, targets):
    raise NotImplementedError("write your pallas kernel here")


# speedup vs baseline: 33.7179x; 33.7179x over previous
import functools

import jax
import jax.numpy as jnp
from jax import lax
from jax.experimental import pallas as pl
from jax.experimental.pallas import tpu as pltpu
from jax.experimental.pallas import tpu_sc as plsc

N_TOTAL = 16 * 1 * 512 * 512
NC, NS, L = 2, 16, 16
NW = NC * NS
P = N_TOTAL // NW
CH = 8192
NCH = P // CH

NBIN = 20480
EXP_LO = 117
TWO_B = 2 * NBIN

BCE_W = 0.8 * 0.5
LOV_W = 0.2 * 1.0


def _sc_stage(x_hbm, t_hbm, cnt_out, sm_out, misc_out, xbuf, tbuf, cnt_v, sm_v,
              mbuf):
    wid = lax.axis_index("s") * NC + lax.axis_index("c")
    base = wid * P

    zeros = jnp.zeros((L,), jnp.float32)

    def zero_body(i, _):
        cnt_v[pl.ds(i * L, L)] = zeros
        sm_v[pl.ds(i * L, L)] = zeros
        return 0

    lax.fori_loop(0, TWO_B // L, zero_body, 0)

    ones = jnp.ones((L,), jnp.float32)

    def vec_body(vi, carry):
        bce_a, k_a, mx_a = carry
        x = xbuf[pl.ds(vi * L, L)]
        t = tbuf[pl.ds(vi * L, L)]
        is_pos = t == 0.0
        e = 1.0 - x * jnp.where(is_pos, 1.0, -1.0)
        relu_e = jnp.maximum(e, 0.0)
        y = jnp.exp(-jnp.abs(x))
        z = y / (2.0 + y)
        z2 = z * z
        lg = 2.0 * z * (1.0 + z2 * (0.33333333 + z2 * (0.2 + z2 * 0.14285714)))
        bce_a = bce_a + (jnp.maximum(x, 0.0) - x * t + lg)
        k_a = k_a + jnp.where(is_pos, 1.0, 0.0)
        mx_a = jnp.maximum(mx_a, e)
        bits = lax.bitcast_convert_type(e, jnp.int32)
        key = jnp.clip((bits >> 13) - (EXP_LO * 1024), 0, NBIN - 1)
        key = key + jnp.where(is_pos, NBIN, 0)
        valid = e > 0.0
        plsc.addupdate_scatter(cnt_v, [key], ones, mask=valid)
        plsc.addupdate_scatter(sm_v, [key], relu_e, mask=valid)
        return (bce_a, k_a, mx_a)

    def chunk_body(ci, carry):
        off = base + ci * CH
        pltpu.sync_copy(x_hbm.at[pl.ds(off, CH)], xbuf)
        pltpu.sync_copy(t_hbm.at[pl.ds(off, CH)], tbuf)
        return lax.fori_loop(0, CH // L, vec_body, carry)

    init = (zeros, zeros, jnp.full((L,), -jnp.inf, jnp.float32))
    bce_a, k_a, mx_a = lax.fori_loop(0, NCH, chunk_body, init)

    mbuf[0, :] = bce_a
    mbuf[1, :] = k_a
    mbuf[2, :] = mx_a
    pltpu.sync_copy(cnt_v, cnt_out.at[wid])
    pltpu.sync_copy(sm_v, sm_out.at[wid])
    pltpu.sync_copy(mbuf, misc_out.at[wid])


_sc_histogram = functools.partial(
    pl.kernel,
    out_type=(
        jax.ShapeDtypeStruct((NW, TWO_B), jnp.float32),
        jax.ShapeDtypeStruct((NW, TWO_B), jnp.float32),
        jax.ShapeDtypeStruct((NW, 3, L), jnp.float32),
    ),
    mesh=plsc.VectorSubcoreMesh(core_axis_name="c", subcore_axis_name="s",
                                num_cores=NC, num_subcores=NS),
    compiler_params=pltpu.CompilerParams(needs_layout_passes=False),
    scratch_types=[
        pltpu.VMEM((CH,), jnp.float32),
        pltpu.VMEM((CH,), jnp.float32),
        pltpu.VMEM((TWO_B,), jnp.float32),
        pltpu.VMEM((TWO_B,), jnp.float32),
        pltpu.VMEM((3, L), jnp.float32),
    ],
)(_sc_stage)


ROWS = NBIN // 128


def _tc_combine(cnt_ref, sm_ref, misc_ref, out_ref):
    cnt = jnp.sum(cnt_ref[...], axis=0)
    sm = jnp.sum(sm_ref[...], axis=0)
    ncnt, pcnt = cnt[:ROWS], cnt[ROWS:]
    nsum, psum = sm[:ROWS], sm[ROWS:]

    ii = lax.broadcasted_iota(jnp.int32, (128, 128), 0)
    jj = lax.broadcasted_iota(jnp.int32, (128, 128), 1)
    tri_incl = (ii <= jj).astype(jnp.float32)
    i2 = lax.broadcasted_iota(jnp.int32, (ROWS, ROWS), 0)
    j2 = lax.broadcasted_iota(jnp.int32, (ROWS, ROWS), 1)
    tri_strict = (j2 < i2).astype(jnp.float32)

    def incl_cumsum(v):
        rowcum = jnp.dot(v, tri_incl, preferred_element_type=jnp.float32,
                         precision=lax.Precision.HIGHEST)
        rowpref = jnp.dot(tri_strict, rowcum[:, 127:128],
                          preferred_element_type=jnp.float32,
                          precision=lax.Precision.HIGHEST)
        return rowcum + rowpref

    s_n = jnp.sum(ncnt) - incl_cumsum(ncnt)
    s_p = jnp.sum(pcnt) - incl_cumsum(pcnt)

    misc = misc_ref[...]
    bce_mean = jnp.sum(misc[:, 0, :]) / N_TOTAL
    kk = jnp.sum(misc[:, 1, :])
    emax = jnp.max(misc[:, 2, :])

    pos_total = jnp.sum(psum / (kk + s_n))
    neg_total = jnp.sum(nsum * (kk - s_p - pcnt) /
                        ((kk + s_n) * (kk + s_n + ncnt)))
    lovasz = jnp.where(kk == 0.0, jnp.maximum(emax, 0.0),
                       pos_total + neg_total)
    out_ref[...] = (BCE_W * bce_mean + LOV_W * lovasz)[None, None]


def kernel(inputs, targets):
    x = inputs.reshape(N_TOTAL)
    t = targets.reshape(N_TOTAL)
    cnt, sm, misc = _sc_histogram(x, t)
    out = pl.pallas_call(
        _tc_combine,
        out_shape=jax.ShapeDtypeStruct((1, 1), jnp.float32),
    )(cnt.reshape(NW, 2 * ROWS, 128), sm.reshape(NW, 2 * ROWS, 128), misc)
    return out.reshape(())


# speedup vs baseline: 36.6784x; 1.0878x over previous
import functools

import jax
import jax.numpy as jnp
from jax import lax
from jax.experimental import pallas as pl
from jax.experimental.pallas import tpu as pltpu
from jax.experimental.pallas import tpu_sc as plsc

N_TOTAL = 16 * 1 * 512 * 512
NC, NS, L = 2, 16, 16
NW = NC * NS
P = N_TOTAL // NW
CH = 8192
NCH = P // CH

NBIN = 20480
EXP_LO = 117
TWO_B = 2 * NBIN

BCE_W = 0.8 * 0.5
LOV_W = 0.2 * 1.0


NBUF = 2


def _sc_stage(x_hbm, t_hbm, cnt_out, sm_out, misc_out, xbuf, tbuf, cnt_v, sm_v,
              mbuf, sems):
    wid = lax.axis_index("s") * NC + lax.axis_index("c")
    base = wid * P

    for b in range(NBUF):
        off = base + b * CH
        pltpu.async_copy(x_hbm.at[pl.ds(off, CH)], xbuf.at[b], sems.at[b])
        pltpu.async_copy(t_hbm.at[pl.ds(off, CH)], tbuf.at[b], sems.at[b])

    zeros = jnp.zeros((L,), jnp.float32)

    def zero_body(i, _):
        cnt_v[pl.ds(i * L, L)] = zeros
        sm_v[pl.ds(i * L, L)] = zeros
        return 0

    lax.fori_loop(0, TWO_B // L, zero_body, 0)

    ones = jnp.ones((L,), jnp.float32)

    def make_vec_body(b):
      def vec_body(vi, carry):
        bce_a, k_a, mx_a = carry
        x = xbuf[b, pl.ds(vi * L, L)]
        t = tbuf[b, pl.ds(vi * L, L)]
        is_pos = t == 0.0
        e = 1.0 - x * jnp.where(is_pos, 1.0, -1.0)
        relu_e = jnp.maximum(e, 0.0)
        y = jnp.exp(-jnp.abs(x))
        z = y / (2.0 + y)
        z2 = z * z
        lg = 2.0 * z * (1.0 + z2 * (0.33333333 + z2 * (0.2 + z2 * 0.14285714)))
        bce_a = bce_a + (jnp.maximum(x, 0.0) - x * t + lg)
        k_a = k_a + jnp.where(is_pos, 1.0, 0.0)
        mx_a = jnp.maximum(mx_a, e)
        bits = lax.bitcast_convert_type(e, jnp.int32)
        key = jnp.clip((bits >> 13) - (EXP_LO * 1024), 0, NBIN - 1)
        key = key + jnp.where(is_pos, NBIN, 0)
        valid = e > 0.0
        plsc.addupdate_scatter(cnt_v, [key], ones, mask=valid)
        plsc.addupdate_scatter(sm_v, [key], relu_e, mask=valid)
        return (bce_a, k_a, mx_a)
      return vec_body

    def pair_body(g, carry):
        for b in range(NBUF):
            ci = g * NBUF + b
            pltpu.make_async_copy(x_hbm.at[pl.ds(0, CH)], xbuf.at[b],
                                  sems.at[b]).wait()
            pltpu.make_async_copy(t_hbm.at[pl.ds(0, CH)], tbuf.at[b],
                                  sems.at[b]).wait()
            carry = lax.fori_loop(0, CH // L, make_vec_body(b), carry)

            @pl.when(ci + NBUF < NCH)
            def _():
                off = base + (ci + NBUF) * CH
                pltpu.async_copy(x_hbm.at[pl.ds(off, CH)], xbuf.at[b],
                                 sems.at[b])
                pltpu.async_copy(t_hbm.at[pl.ds(off, CH)], tbuf.at[b],
                                 sems.at[b])
        return carry

    init = (zeros, zeros, jnp.full((L,), -jnp.inf, jnp.float32))
    bce_a, k_a, mx_a = lax.fori_loop(0, NCH // NBUF, pair_body, init)

    mbuf[0, :] = bce_a
    mbuf[1, :] = k_a
    mbuf[2, :] = mx_a
    pltpu.sync_copy(cnt_v, cnt_out.at[wid])
    pltpu.sync_copy(sm_v, sm_out.at[wid])
    pltpu.sync_copy(mbuf, misc_out.at[wid])


_sc_histogram = functools.partial(
    pl.kernel,
    out_type=(
        jax.ShapeDtypeStruct((NW, TWO_B), jnp.float32),
        jax.ShapeDtypeStruct((NW, TWO_B), jnp.float32),
        jax.ShapeDtypeStruct((NW, 3, L), jnp.float32),
    ),
    mesh=plsc.VectorSubcoreMesh(core_axis_name="c", subcore_axis_name="s",
                                num_cores=NC, num_subcores=NS),
    compiler_params=pltpu.CompilerParams(needs_layout_passes=False),
    scratch_types=[
        pltpu.VMEM((NBUF, CH), jnp.float32),
        pltpu.VMEM((NBUF, CH), jnp.float32),
        pltpu.VMEM((TWO_B,), jnp.float32),
        pltpu.VMEM((TWO_B,), jnp.float32),
        pltpu.VMEM((3, L), jnp.float32),
        pltpu.SemaphoreType.DMA((NBUF,)),
    ],
)(_sc_stage)


ROWS = NBIN // 128


def _tc_combine(cnt_ref, sm_ref, misc_ref, out_ref):
    cnt = jnp.sum(cnt_ref[...], axis=0)
    sm = jnp.sum(sm_ref[...], axis=0)
    ncnt, pcnt = cnt[:ROWS], cnt[ROWS:]
    nsum, psum = sm[:ROWS], sm[ROWS:]

    ii = lax.broadcasted_iota(jnp.int32, (128, 128), 0)
    jj = lax.broadcasted_iota(jnp.int32, (128, 128), 1)
    tri_incl = (ii <= jj).astype(jnp.float32)
    i2 = lax.broadcasted_iota(jnp.int32, (ROWS, ROWS), 0)
    j2 = lax.broadcasted_iota(jnp.int32, (ROWS, ROWS), 1)
    tri_strict = (j2 < i2).astype(jnp.float32)

    def incl_cumsum(v):
        rowcum = jnp.dot(v, tri_incl, preferred_element_type=jnp.float32,
                         precision=lax.Precision.HIGHEST)
        rowpref = jnp.dot(tri_strict, rowcum[:, 127:128],
                          preferred_element_type=jnp.float32,
                          precision=lax.Precision.HIGHEST)
        return rowcum + rowpref

    s_n = jnp.sum(ncnt) - incl_cumsum(ncnt)
    s_p = jnp.sum(pcnt) - incl_cumsum(pcnt)

    misc = misc_ref[...]
    bce_mean = jnp.sum(misc[:, 0, :]) / N_TOTAL
    kk = jnp.sum(misc[:, 1, :])
    emax = jnp.max(misc[:, 2, :])

    pos_total = jnp.sum(psum / (kk + s_n))
    neg_total = jnp.sum(nsum * (kk - s_p - pcnt) /
                        ((kk + s_n) * (kk + s_n + ncnt)))
    lovasz = jnp.where(kk == 0.0, jnp.maximum(emax, 0.0),
                       pos_total + neg_total)
    out_ref[...] = (BCE_W * bce_mean + LOV_W * lovasz)[None, None]


def kernel(inputs, targets):
    x = inputs.reshape(N_TOTAL)
    t = targets.reshape(N_TOTAL)
    cnt, sm, misc = _sc_histogram(x, t)
    out = pl.pallas_call(
        _tc_combine,
        out_shape=jax.ShapeDtypeStruct((1, 1), jnp.float32),
    )(cnt.reshape(NW, 2 * ROWS, 128), sm.reshape(NW, 2 * ROWS, 128), misc)
    return out.reshape(())


# speedup vs baseline: 38.1944x; 1.0413x over previous
import functools

import jax
import jax.numpy as jnp
from jax import lax
from jax.experimental import pallas as pl
from jax.experimental.pallas import tpu as pltpu
from jax.experimental.pallas import tpu_sc as plsc

N_TOTAL = 16 * 1 * 512 * 512
NC, NS, L = 2, 16, 16
NW = NC * NS
P = N_TOTAL // NW
CH = 8192
NCH = P // CH
NBUF = 2

NBIN = 20480
EXP_LO = 117
TWO_B = 2 * NBIN

BCE_W = 0.8 * 0.5
LOV_W = 0.2 * 1.0


def _sc_stage(x_hbm, t_hbm, cnt_out, sm_out, xbuf, tbuf, cnt_v, sm_v, sems):
    wid = lax.axis_index("s") * NC + lax.axis_index("c")
    base = wid * P

    for b in range(NBUF):
        off = base + b * CH
        pltpu.async_copy(x_hbm.at[pl.ds(off, CH)], xbuf.at[b], sems.at[b])
        pltpu.async_copy(t_hbm.at[pl.ds(off, CH)], tbuf.at[b], sems.at[b])

    zeros = jnp.zeros((L,), jnp.float32)

    def zero_body(i, _):
        cnt_v[pl.ds(i * L, L)] = zeros
        sm_v[pl.ds(i * L, L)] = zeros
        return 0

    lax.fori_loop(0, TWO_B // L, zero_body, 0)

    ones = jnp.ones((L,), jnp.float32)

    def make_vec_body(b):
      def vec_body(vi, carry):
        x = xbuf[b, pl.ds(vi * L, L)]
        t = tbuf[b, pl.ds(vi * L, L)]
        is_pos = t == 0.0
        e = 1.0 - x * jnp.where(is_pos, 1.0, -1.0)
        relu_e = jnp.maximum(e, 0.0)
        bits = lax.bitcast_convert_type(e, jnp.int32)
        key = jnp.clip((bits >> 13) - (EXP_LO * 1024), 0, NBIN - 1)
        key = key + jnp.where(is_pos, NBIN, 0)
        valid = (e > 0.0) | is_pos
        plsc.addupdate_scatter(cnt_v, [key], ones, mask=valid)
        plsc.addupdate_scatter(sm_v, [key], relu_e, mask=valid)
        return carry
      return vec_body

    def pair_body(g, carry):
        for b in range(NBUF):
            ci = g * NBUF + b
            pltpu.make_async_copy(x_hbm.at[pl.ds(0, CH)], xbuf.at[b],
                                  sems.at[b]).wait()
            pltpu.make_async_copy(t_hbm.at[pl.ds(0, CH)], tbuf.at[b],
                                  sems.at[b]).wait()
            carry = lax.fori_loop(0, CH // L, make_vec_body(b), carry)

            @pl.when(ci + NBUF < NCH)
            def _():
                off = base + (ci + NBUF) * CH
                pltpu.async_copy(x_hbm.at[pl.ds(off, CH)], xbuf.at[b],
                                 sems.at[b])
                pltpu.async_copy(t_hbm.at[pl.ds(off, CH)], tbuf.at[b],
                                 sems.at[b])
        return carry

    lax.fori_loop(0, NCH // NBUF, pair_body, 0)

    pltpu.sync_copy(cnt_v, cnt_out.at[wid])
    pltpu.sync_copy(sm_v, sm_out.at[wid])


_sc_histogram = functools.partial(
    pl.kernel,
    out_type=(
        jax.ShapeDtypeStruct((NW, TWO_B), jnp.float32),
        jax.ShapeDtypeStruct((NW, TWO_B), jnp.float32),
    ),
    mesh=plsc.VectorSubcoreMesh(core_axis_name="c", subcore_axis_name="s",
                                num_cores=NC, num_subcores=NS),
    compiler_params=pltpu.CompilerParams(needs_layout_passes=False),
    scratch_types=[
        pltpu.VMEM((NBUF, CH), jnp.float32),
        pltpu.VMEM((NBUF, CH), jnp.float32),
        pltpu.VMEM((TWO_B,), jnp.float32),
        pltpu.VMEM((TWO_B,), jnp.float32),
        pltpu.SemaphoreType.DMA((NBUF,)),
    ],
)(_sc_stage)


BCE_GRID = 8
BCE_ROWS = (N_TOTAL // 512) // BCE_GRID


def _tc_bce(x_ref, t_ref, out_ref):
    x = x_ref[...]
    t = t_ref[...]
    bce = jnp.maximum(x, 0.0) - x * t + jnp.log1p(jnp.exp(-jnp.abs(x)))
    part = jnp.sum(bce)

    @pl.when(pl.program_id(0) == 0)
    def _():
        out_ref[...] = jnp.zeros_like(out_ref)

    out_ref[...] += part[None, None]


ROWS = NBIN // 128


def _tc_combine(cnt_ref, sm_ref, bce_ref, out_ref):
    cnt = jnp.sum(cnt_ref[...], axis=0)
    sm = jnp.sum(sm_ref[...], axis=0)
    ncnt, pcnt = cnt[:ROWS], cnt[ROWS:]
    nsum, psum = sm[:ROWS], sm[ROWS:]

    ii = lax.broadcasted_iota(jnp.int32, (128, 128), 0)
    jj = lax.broadcasted_iota(jnp.int32, (128, 128), 1)
    tri_incl = (ii <= jj).astype(jnp.float32)
    i2 = lax.broadcasted_iota(jnp.int32, (ROWS, ROWS), 0)
    j2 = lax.broadcasted_iota(jnp.int32, (ROWS, ROWS), 1)
    tri_strict = (j2 < i2).astype(jnp.float32)

    def incl_cumsum(v):
        rowcum = jnp.dot(v, tri_incl, preferred_element_type=jnp.float32,
                         precision=lax.Precision.HIGHEST)
        rowpref = jnp.dot(tri_strict, rowcum[:, 127:128],
                          preferred_element_type=jnp.float32,
                          precision=lax.Precision.HIGHEST)
        return rowcum + rowpref

    s_n = jnp.sum(ncnt) - incl_cumsum(ncnt)
    s_p = jnp.sum(pcnt) - incl_cumsum(pcnt)

    kk = jnp.sum(pcnt)
    bce_mean = bce_ref[0, 0] / N_TOTAL
    emax = jnp.max(jnp.where(ncnt > 0.0, nsum / jnp.maximum(ncnt, 1.0),
                             -jnp.inf))

    pos_total = jnp.sum(psum / (kk + s_n))
    neg_total = jnp.sum(nsum * (kk - s_p - pcnt) /
                        ((kk + s_n) * (kk + s_n + ncnt)))
    lovasz = jnp.where(kk == 0.0, jnp.maximum(emax, 0.0),
                       pos_total + neg_total)
    out_ref[...] = (BCE_W * bce_mean + LOV_W * lovasz)[None, None]


def kernel(inputs, targets):
    x = inputs.reshape(N_TOTAL)
    t = targets.reshape(N_TOTAL)
    cnt, sm = _sc_histogram(x, t)
    bce = pl.pallas_call(
        _tc_bce,
        grid=(BCE_GRID,),
        in_specs=[
            pl.BlockSpec((BCE_ROWS, 512), lambda i: (i, 0)),
            pl.BlockSpec((BCE_ROWS, 512), lambda i: (i, 0)),
        ],
        out_specs=pl.BlockSpec((1, 1), lambda i: (0, 0)),
        out_shape=jax.ShapeDtypeStruct((1, 1), jnp.float32),
    )(inputs.reshape(N_TOTAL // 512, 512), targets.reshape(N_TOTAL // 512, 512))
    out = pl.pallas_call(
        _tc_combine,
        out_shape=jax.ShapeDtypeStruct((1, 1), jnp.float32),
    )(cnt.reshape(NW, 2 * ROWS, 128), sm.reshape(NW, 2 * ROWS, 128), bce)
    return out.reshape(())


# speedup vs baseline: 60.9240x; 1.5951x over previous
import functools

import jax
import jax.numpy as jnp
from jax import lax
from jax.experimental import pallas as pl
from jax.experimental.pallas import tpu as pltpu
from jax.experimental.pallas import tpu_sc as plsc

N_TOTAL = 16 * 1 * 512 * 512
NC, NS, L = 2, 16, 16
NW = NC * NS
P = N_TOTAL // NW
CH = 8192
NCH = P // CH
NBUF = 2

NBIN = 20480
EXP_LO = 117
TWO_B = 2 * NBIN

BCE_W = 0.8 * 0.5
LOV_W = 0.2 * 1.0


def _sc_stage(x_hbm, t_hbm, cnt_out, sm_out, xbuf, tbuf, cnt_v, sm_v, sems):
    wid = lax.axis_index("s") * NC + lax.axis_index("c")
    base = wid * P

    for b in range(NBUF):
        off = base + b * CH
        pltpu.async_copy(x_hbm.at[pl.ds(off, CH)], xbuf.at[b], sems.at[b])
        pltpu.async_copy(t_hbm.at[pl.ds(off, CH)], tbuf.at[b], sems.at[b])

    zeros = jnp.zeros((L,), jnp.float32)

    def zero_body(i, _):
        cnt_v[pl.ds(i * L, L)] = zeros
        sm_v[pl.ds(i * L, L)] = zeros
        return 0

    lax.fori_loop(0, TWO_B // L, zero_body, 0)

    ones = jnp.ones((L,), jnp.float32)

    def process_chunk(b):
        @plsc.parallel_loop(0, CH, L, unroll=8)
        def _(i):
            x = xbuf[b, pl.ds(i, L)]
            t = tbuf[b, pl.ds(i, L)]
            is_pos = t == 0.0
            e = 1.0 - x * jnp.where(is_pos, 1.0, -1.0)
            relu_e = jnp.maximum(e, 0.0)
            bits = lax.bitcast_convert_type(e, jnp.int32)
            key = jnp.clip((bits >> 13) - (EXP_LO * 1024), 0, NBIN - 1)
            key = key + jnp.where(is_pos, NBIN, 0)
            valid = (e > 0.0) | is_pos
            plsc.addupdate_scatter(cnt_v, [key], ones, mask=valid)
            plsc.addupdate_scatter(sm_v, [key], relu_e, mask=valid)

    def pair_body(g, carry):
        for b in range(NBUF):
            ci = g * NBUF + b
            pltpu.make_async_copy(x_hbm.at[pl.ds(0, CH)], xbuf.at[b],
                                  sems.at[b]).wait()
            pltpu.make_async_copy(t_hbm.at[pl.ds(0, CH)], tbuf.at[b],
                                  sems.at[b]).wait()
            process_chunk(b)

            @pl.when(ci + NBUF < NCH)
            def _():
                off = base + (ci + NBUF) * CH
                pltpu.async_copy(x_hbm.at[pl.ds(off, CH)], xbuf.at[b],
                                 sems.at[b])
                pltpu.async_copy(t_hbm.at[pl.ds(off, CH)], tbuf.at[b],
                                 sems.at[b])
        return carry

    lax.fori_loop(0, NCH // NBUF, pair_body, 0)

    pltpu.sync_copy(cnt_v, cnt_out.at[wid])
    pltpu.sync_copy(sm_v, sm_out.at[wid])


_sc_histogram = functools.partial(
    pl.kernel,
    out_type=(
        jax.ShapeDtypeStruct((NW, TWO_B), jnp.float32),
        jax.ShapeDtypeStruct((NW, TWO_B), jnp.float32),
    ),
    mesh=plsc.VectorSubcoreMesh(core_axis_name="c", subcore_axis_name="s",
                                num_cores=NC, num_subcores=NS),
    compiler_params=pltpu.CompilerParams(needs_layout_passes=False),
    scratch_types=[
        pltpu.VMEM((NBUF, CH), jnp.float32),
        pltpu.VMEM((NBUF, CH), jnp.float32),
        pltpu.VMEM((TWO_B,), jnp.float32),
        pltpu.VMEM((TWO_B,), jnp.float32),
        pltpu.SemaphoreType.DMA((NBUF,)),
    ],
)(_sc_stage)


BCE_GRID = 8
BCE_ROWS = (N_TOTAL // 512) // BCE_GRID


def _tc_bce(x_ref, t_ref, out_ref):
    x = x_ref[...]
    t = t_ref[...]
    bce = jnp.maximum(x, 0.0) - x * t + jnp.log1p(jnp.exp(-jnp.abs(x)))
    part = jnp.sum(bce)

    @pl.when(pl.program_id(0) == 0)
    def _():
        out_ref[...] = jnp.zeros_like(out_ref)

    out_ref[...] += part[None, None]


ROWS = NBIN // 128


def _tc_combine(cnt_ref, sm_ref, bce_ref, out_ref):
    cnt = jnp.sum(cnt_ref[...], axis=0)
    sm = jnp.sum(sm_ref[...], axis=0)
    ncnt, pcnt = cnt[:ROWS], cnt[ROWS:]
    nsum, psum = sm[:ROWS], sm[ROWS:]

    ii = lax.broadcasted_iota(jnp.int32, (128, 128), 0)
    jj = lax.broadcasted_iota(jnp.int32, (128, 128), 1)
    tri_incl = (ii <= jj).astype(jnp.float32)
    i2 = lax.broadcasted_iota(jnp.int32, (ROWS, ROWS), 0)
    j2 = lax.broadcasted_iota(jnp.int32, (ROWS, ROWS), 1)
    tri_strict = (j2 < i2).astype(jnp.float32)

    def incl_cumsum(v):
        rowcum = jnp.dot(v, tri_incl, preferred_element_type=jnp.float32,
                         precision=lax.Precision.HIGHEST)
        rowpref = jnp.dot(tri_strict, rowcum[:, 127:128],
                          preferred_element_type=jnp.float32,
                          precision=lax.Precision.HIGHEST)
        return rowcum + rowpref

    s_n = jnp.sum(ncnt) - incl_cumsum(ncnt)
    s_p = jnp.sum(pcnt) - incl_cumsum(pcnt)

    kk = jnp.sum(pcnt)
    bce_mean = bce_ref[0, 0] / N_TOTAL
    emax = jnp.max(jnp.where(ncnt > 0.0, nsum / jnp.maximum(ncnt, 1.0),
                             -jnp.inf))

    pos_total = jnp.sum(psum / (kk + s_n))
    neg_total = jnp.sum(nsum * (kk - s_p - pcnt) /
                        ((kk + s_n) * (kk + s_n + ncnt)))
    lovasz = jnp.where(kk == 0.0, jnp.maximum(emax, 0.0),
                       pos_total + neg_total)
    out_ref[...] = (BCE_W * bce_mean + LOV_W * lovasz)[None, None]


def kernel(inputs, targets):
    x = inputs.reshape(N_TOTAL)
    t = targets.reshape(N_TOTAL)
    cnt, sm = _sc_histogram(x, t)
    bce = pl.pallas_call(
        _tc_bce,
        grid=(BCE_GRID,),
        in_specs=[
            pl.BlockSpec((BCE_ROWS, 512), lambda i: (i, 0)),
            pl.BlockSpec((BCE_ROWS, 512), lambda i: (i, 0)),
        ],
        out_specs=pl.BlockSpec((1, 1), lambda i: (0, 0)),
        out_shape=jax.ShapeDtypeStruct((1, 1), jnp.float32),
    )(inputs.reshape(N_TOTAL // 512, 512), targets.reshape(N_TOTAL // 512, 512))
    out = pl.pallas_call(
        _tc_combine,
        out_shape=jax.ShapeDtypeStruct((1, 1), jnp.float32),
    )(cnt.reshape(NW, 2 * ROWS, 128), sm.reshape(NW, 2 * ROWS, 128), bce)
    return out.reshape(())


# speedup vs baseline: 63.8691x; 1.0483x over previous
import functools

import jax
import jax.numpy as jnp
from jax import lax
from jax.experimental import pallas as pl
from jax.experimental.pallas import tpu as pltpu
from jax.experimental.pallas import tpu_sc as plsc

N_TOTAL = 16 * 1 * 512 * 512
NC, NS, L = 2, 16, 16
NW = NC * NS
P = N_TOTAL // NW
CH = 8192
NCH = P // CH
NBUF = 2

NBIN = 20480
EXP_LO = 117
TWO_B = 2 * NBIN

BCE_W = 0.8 * 0.5
LOV_W = 0.2 * 1.0


def _sc_stage(x_hbm, t_hbm, cnt_out, sm_out, xbuf, tbuf, cnt_v, sm_v, sems):
    wid = lax.axis_index("s") * NC + lax.axis_index("c")
    base = wid * P

    for b in range(NBUF):
        off = base + b * CH
        pltpu.async_copy(x_hbm.at[pl.ds(off, CH)], xbuf.at[b], sems.at[b])
        pltpu.async_copy(t_hbm.at[pl.ds(off, CH)], tbuf.at[b], sems.at[b])

    zeros = jnp.zeros((L,), jnp.float32)

    @plsc.parallel_loop(0, TWO_B, L, unroll=8)
    def _(i):
        cnt_v[pl.ds(i, L)] = zeros
        sm_v[pl.ds(i, L)] = zeros

    ones = jnp.ones((L,), jnp.float32)

    def process_chunk(b):
        @plsc.parallel_loop(0, CH, L, unroll=8)
        def _(i):
            x = xbuf[b, pl.ds(i, L)]
            t = tbuf[b, pl.ds(i, L)]
            is_pos = t == 0.0
            e = 1.0 - x * jnp.where(is_pos, 1.0, -1.0)
            relu_e = jnp.maximum(e, 0.0)
            bits = lax.bitcast_convert_type(e, jnp.int32)
            key = jnp.clip((bits >> 13) - (EXP_LO * 1024), 0, NBIN - 1)
            key = key + jnp.where(is_pos, NBIN, 0)
            valid = (e > 0.0) | is_pos
            plsc.addupdate_scatter(cnt_v, [key], ones, mask=valid)
            plsc.addupdate_scatter(sm_v, [key], relu_e, mask=valid)

    def pair_body(g, carry):
        for b in range(NBUF):
            ci = g * NBUF + b
            pltpu.make_async_copy(x_hbm.at[pl.ds(0, CH)], xbuf.at[b],
                                  sems.at[b]).wait()
            pltpu.make_async_copy(t_hbm.at[pl.ds(0, CH)], tbuf.at[b],
                                  sems.at[b]).wait()
            process_chunk(b)

            @pl.when(ci + NBUF < NCH)
            def _():
                off = base + (ci + NBUF) * CH
                pltpu.async_copy(x_hbm.at[pl.ds(off, CH)], xbuf.at[b],
                                 sems.at[b])
                pltpu.async_copy(t_hbm.at[pl.ds(off, CH)], tbuf.at[b],
                                 sems.at[b])
        return carry

    lax.fori_loop(0, NCH // NBUF, pair_body, 0)

    pltpu.sync_copy(cnt_v, cnt_out.at[wid])
    pltpu.sync_copy(sm_v, sm_out.at[wid])


_sc_histogram = functools.partial(
    pl.kernel,
    out_type=(
        jax.ShapeDtypeStruct((NW, TWO_B), jnp.float32),
        jax.ShapeDtypeStruct((NW, TWO_B), jnp.float32),
    ),
    mesh=plsc.VectorSubcoreMesh(core_axis_name="c", subcore_axis_name="s",
                                num_cores=NC, num_subcores=NS),
    compiler_params=pltpu.CompilerParams(needs_layout_passes=False),
    scratch_types=[
        pltpu.VMEM((NBUF, CH), jnp.float32),
        pltpu.VMEM((NBUF, CH), jnp.float32),
        pltpu.VMEM((TWO_B,), jnp.float32),
        pltpu.VMEM((TWO_B,), jnp.float32),
        pltpu.SemaphoreType.DMA((NBUF,)),
    ],
)(_sc_stage)


BCE_GRID = 8
BCE_ROWS = (N_TOTAL // 512) // BCE_GRID


def _tc_bce(x_ref, t_ref, out_ref):
    x = x_ref[...]
    t = t_ref[...]
    bce = jnp.maximum(x, 0.0) - x * t + jnp.log1p(jnp.exp(-jnp.abs(x)))
    part = jnp.sum(bce)

    @pl.when(pl.program_id(0) == 0)
    def _():
        out_ref[...] = jnp.zeros_like(out_ref)

    out_ref[...] += part[None, None]


ROWS = NBIN // 128


def _tc_combine(cnt_ref, sm_ref, bce_ref, out_ref):
    cnt = jnp.sum(cnt_ref[...], axis=0)
    sm = jnp.sum(sm_ref[...], axis=0)
    ncnt, pcnt = cnt[:ROWS], cnt[ROWS:]
    nsum, psum = sm[:ROWS], sm[ROWS:]

    ii = lax.broadcasted_iota(jnp.int32, (128, 128), 0)
    jj = lax.broadcasted_iota(jnp.int32, (128, 128), 1)
    tri_incl = (ii <= jj).astype(jnp.float32)
    i2 = lax.broadcasted_iota(jnp.int32, (ROWS, ROWS), 0)
    j2 = lax.broadcasted_iota(jnp.int32, (ROWS, ROWS), 1)
    tri_strict = (j2 < i2).astype(jnp.float32)

    def incl_cumsum(v):
        rowcum = jnp.dot(v, tri_incl, preferred_element_type=jnp.float32,
                         precision=lax.Precision.HIGHEST)
        rowpref = jnp.dot(tri_strict, rowcum[:, 127:128],
                          preferred_element_type=jnp.float32,
                          precision=lax.Precision.HIGHEST)
        return rowcum + rowpref

    s_n = jnp.sum(ncnt) - incl_cumsum(ncnt)
    s_p = jnp.sum(pcnt) - incl_cumsum(pcnt)

    kk = jnp.sum(pcnt)
    bce_mean = bce_ref[0, 0] / N_TOTAL
    emax = jnp.max(jnp.where(ncnt > 0.0, nsum / jnp.maximum(ncnt, 1.0),
                             -jnp.inf))

    pos_total = jnp.sum(psum / (kk + s_n))
    neg_total = jnp.sum(nsum * (kk - s_p - pcnt) /
                        ((kk + s_n) * (kk + s_n + ncnt)))
    lovasz = jnp.where(kk == 0.0, jnp.maximum(emax, 0.0),
                       pos_total + neg_total)
    out_ref[...] = (BCE_W * bce_mean + LOV_W * lovasz)[None, None]


def kernel(inputs, targets):
    x = inputs.reshape(N_TOTAL)
    t = targets.reshape(N_TOTAL)
    bce = pl.pallas_call(
        _tc_bce,
        grid=(BCE_GRID,),
        in_specs=[
            pl.BlockSpec((BCE_ROWS, 512), lambda i: (i, 0)),
            pl.BlockSpec((BCE_ROWS, 512), lambda i: (i, 0)),
        ],
        out_specs=pl.BlockSpec((1, 1), lambda i: (0, 0)),
        out_shape=jax.ShapeDtypeStruct((1, 1), jnp.float32),
    )(inputs.reshape(N_TOTAL // 512, 512), targets.reshape(N_TOTAL // 512, 512))
    cnt, sm = _sc_histogram(x, t)
    out = pl.pallas_call(
        _tc_combine,
        out_shape=jax.ShapeDtypeStruct((1, 1), jnp.float32),
    )(cnt.reshape(NW, 2 * ROWS, 128), sm.reshape(NW, 2 * ROWS, 128), bce)
    return out.reshape(())
